# baseline (speedup 1.0000x reference)
import jax
import jax.numpy as jnp
from jax import lax
from jax.experimental import pallas as pl
from jax.experimental.pallas import tpu as pltpu
from jax.experimental.pallas import tpu_sc as plsc

N = 1024
C = 192
CP = 256
HC = 32
WC = 32
M = HC * WC
R = 512
GRID_STEPS = N // R

NW = 32
KPW = N // NW
NCH = C // 16

_BIGF = 1e30
_BIGI = 2**30


def _axis3(v16):
    c0 = (v16 * (1.0 / 16.0)).astype(jnp.int32)
    c0f = c0.astype(jnp.float32)
    d0 = v16 - (16.0 * c0f + 8.0)
    side = jnp.where(d0 >= 0.0, 1, -1).astype(jnp.int32)
    c1 = c0 + side
    c1_in = (c1 >= 0) & (c1 <= HC - 1)
    c1 = jnp.where(c1_in, c1, c0 - side)
    cm = c0 - side
    cm_in = (cm >= 0) & (cm <= HC - 1)
    c2 = jnp.where(cm_in, jnp.where(c1_in, cm, c0 - 2 * side), c0 + 2 * side)

    def d2(ci):
        dd = v16 - (16.0 * ci.astype(jnp.float32) + 8.0)
        return dd * dd

    return [(c0, d2(c0)), (c1, d2(c1)), (c2, d2(c2))]


def _floor16(v16):
    t = v16.astype(jnp.int32).astype(jnp.float32)
    return jnp.where(t > v16, t - 1.0, t)


def _sc_body(x_hbm, y_hbm, d2fp_hbm, wsum_hbm, gidx_hbm,
             x_v, y_v, ws_v, idx_v, wt_v, rows_v, gidx_v, sem):
    wid = lax.axis_index("s") * 2 + lax.axis_index("c")
    base = wid * KPW

    pltpu.sync_copy(x_hbm.at[pl.ds(base, KPW)], x_v)
    pltpu.sync_copy(y_hbm.at[pl.ds(base, KPW)], y_v)

    for h in range(2):
        x16 = x_v[pl.ds(h * 16, 16)]
        y16 = y_v[pl.ds(h * 16, 16)]

        cxs = _axis3(x16)
        cys = _axis3(y16)
        cand = []
        for cyi, dy2 in cys:
            for cxi, dx2 in cxs:
                cand.append((dx2 + dy2, cyi * WC + cxi))
        for k in range(4):
            m = cand[0][0]
            for dcur, _ in cand[1:]:
                m = jnp.minimum(m, dcur)
            f = jnp.full((16,), _BIGI, jnp.int32)
            for dcur, fcur in cand:
                f = jnp.minimum(f, jnp.where(dcur == m, fcur, _BIGI))
            gidx_v[pl.ds(k * KPW + h * 16, 16)] = f
            cand = [(jnp.where(fcur == f, jnp.float32(_BIGF), dcur), fcur)
                    for dcur, fcur in cand]

        ysn = y16 * (1.0 / 16.0) - 0.5
        xsn = x16 * (1.0 / 16.0) - 0.5
        x0 = _floor16(xsn)
        y0 = _floor16(ysn)
        wx1 = xsn - x0
        wx0 = 1.0 - wx1
        wy1 = ysn - y0
        wy0 = 1.0 - wy1
        corners = [(y0, x0, wy0 * wx0), (y0, x0 + 1.0, wy0 * wx1),
                   (y0 + 1.0, x0, wy1 * wx0), (y0 + 1.0, x0 + 1.0, wy1 * wx1)]
        for c, (yi, xi, w) in enumerate(corners):
            valid = ((yi >= 0.0) & (yi <= HC - 1.0)
                     & (xi >= 0.0) & (xi <= WC - 1.0))
            yc = jnp.minimum(jnp.maximum(yi, 0.0), HC - 1.0).astype(jnp.int32)
            xc = jnp.minimum(jnp.maximum(xi, 0.0), WC - 1.0).astype(jnp.int32)
            wt_v[pl.ds(c * KPW + h * 16, 16)] = jnp.where(valid, w, 0.0)
            idx_v[pl.ds(c * KPW + h * 16, 16)] = yc * WC + xc

    pltpu.async_copy(d2fp_hbm.at[idx_v], rows_v, sem).wait()

    def kp_body(k, carry):
        wb = []
        for c in range(4):
            wv = wt_v[pl.ds(c * KPW + k, 16)]
            wb.append(jnp.full((16,), 0.0, jnp.float32) + wv[0])
        for j in range(NCH):
            s = wb[0] * rows_v[k, pl.ds(j * 16, 16)]
            for c in range(1, 4):
                s = s + wb[c] * rows_v[c * KPW + k, pl.ds(j * 16, 16)]
            ws_v[pl.ds(k * C + j * 16, 16)] = s
        return carry

    lax.fori_loop(0, KPW, kp_body, 0)

    pltpu.sync_copy(ws_v, wsum_hbm.at[pl.ds(base * C, KPW * C)])
    for k in range(4):
        pltpu.sync_copy(gidx_v.at[pl.ds(k * KPW, KPW)],
                        gidx_hbm.at[pl.ds(k * N + base, KPW)])


_sc_call = pl.kernel(
    _sc_body,
    mesh=plsc.VectorSubcoreMesh(core_axis_name="c", subcore_axis_name="s"),
    out_type=[
        jax.ShapeDtypeStruct((N * C,), jnp.float32),
        jax.ShapeDtypeStruct((4 * N,), jnp.int32),
    ],
    scratch_types=[
        pltpu.VMEM((KPW,), jnp.float32),
        pltpu.VMEM((KPW,), jnp.float32),
        pltpu.VMEM((KPW * C,), jnp.float32),
        pltpu.VMEM((4 * KPW,), jnp.int32),
        pltpu.VMEM((4 * KPW + 16,), jnp.float32),
        pltpu.VMEM((4 * KPW, CP), jnp.float32),
        pltpu.VMEM((4 * KPW,), jnp.int32),
        pltpu.SemaphoreType.DMA,
    ],
)


def _tc_body(wsum_ref, gidx_ref, kpd_ref, d2ft_ref, loss_ref, pos_out_ref):
    step = pl.program_id(0)
    A = kpd_ref[...]
    W = wsum_ref[...]
    BT = d2ft_ref[...]

    dot = jnp.sum(A * W, axis=1, keepdims=True)
    nrm = jnp.sqrt(jnp.sum(W * W, axis=1, keepdims=True))
    pos = 2.0 - 2.0 * dot / jnp.maximum(nrm, 1e-12)

    cols = jax.lax.broadcasted_iota(jnp.int32, (R, M), 1)

    S = 2.0 - 2.0 * jax.lax.dot_general(
        A, BT, (((1,), (0,)), ((), ())),
        preferred_element_type=jnp.float32)

    for k in range(4):
        S = jnp.where(cols == gidx_ref[k, 0], _BIGF, S)

    m1 = S[:, 0:128]
    m2 = jnp.full((R, 128), _BIGF, jnp.float32)
    for g in range(1, 8):
        Sg = S[:, g * 128:(g + 1) * 128]
        m2 = jnp.minimum(m2, jnp.maximum(m1, Sg))
        m1 = jnp.minimum(m1, Sg)

    acc = jnp.float32(0.0)
    for _ in range(4):
        mm = jnp.minimum(jnp.min(m1, axis=1, keepdims=True),
                         jnp.min(m2, axis=1, keepdims=True))
        acc = acc + jnp.sum(jnp.maximum(pos - mm + 1.0, 0.0))
        m1 = jnp.where(m1 == mm, _BIGF, m1)
        m2 = jnp.where(m2 == mm, _BIGF, m2)

    possum = jnp.sum(pos)

    @pl.when(step == 0)
    def _():
        loss_ref[...] = jnp.zeros((1, 1), jnp.float32)
        pos_out_ref[...] = jnp.zeros((1, 1), jnp.float32)

    loss_ref[...] += jnp.reshape(acc, (1, 1))
    pos_out_ref[...] += jnp.reshape(possum, (1, 1))

    @pl.when(step == GRID_STEPS - 1)
    def _():
        loss_ref[...] = loss_ref[...] * (1.0 / (4.0 * N))
        pos_out_ref[...] = pos_out_ref[...] * (1.0 / N)


@jax.jit
def _run(x, y, kp1_desc, d2fp, d2ft):
    wsum, gidx = _sc_call(x, y, d2fp)
    loss, posmean = pl.pallas_call(
        _tc_body,
        grid=(GRID_STEPS,),
        in_specs=[
            pl.BlockSpec((R, C), lambda i: (i, 0)),
            pl.BlockSpec((4, 1, R, 1), lambda i: (0, i, 0, 0)),
            pl.BlockSpec((R, C), lambda i: (i, 0)),
            pl.BlockSpec((C, M), lambda i: (0, 0)),
        ],
        out_specs=[
            pl.BlockSpec((1, 1), lambda i: (0, 0)),
            pl.BlockSpec((1, 1), lambda i: (0, 0)),
        ],
        out_shape=[
            jax.ShapeDtypeStruct((1, 1), jnp.float32),
            jax.ShapeDtypeStruct((1, 1), jnp.float32),
        ],
    )(wsum.reshape(N, C), gidx.reshape(4, GRID_STEPS, R, 1), kp1_desc, d2ft)
    return loss[0, 0], posmean[0, 0]


def kernel(kp1, w_kp1, kp1_desc, desc2):
    d2ft = desc2[0].reshape(C, M)
    d2f = jnp.transpose(d2ft, (1, 0))
    d2fp = jnp.pad(d2f, ((0, 0), (0, CP - C)))
    x = w_kp1[:, 1]
    y = w_kp1[:, 0]
    return _run(x, y, kp1_desc, d2fp, d2ft)


# speedup vs baseline: 1.0588x; 1.0588x over previous
import jax
import jax.numpy as jnp
from jax import lax
from jax.experimental import pallas as pl
from jax.experimental.pallas import tpu as pltpu
from jax.experimental.pallas import tpu_sc as plsc

N = 1024
C = 192
CP = 256
HC = 32
WC = 32
M = HC * WC
R = 512
GRID_STEPS = N // R

NW = 32
KPW = N // NW
NCH = C // 16

_BIGF = 1e30
_BIGI = 2**30


def _axis3(v16):
    c0 = (v16 * (1.0 / 16.0)).astype(jnp.int32)
    c0f = c0.astype(jnp.float32)
    d0 = v16 - (16.0 * c0f + 8.0)
    side = jnp.where(d0 >= 0.0, 1, -1).astype(jnp.int32)
    c1 = c0 + side
    c1_in = (c1 >= 0) & (c1 <= HC - 1)
    c1 = jnp.where(c1_in, c1, c0 - side)
    cm = c0 - side
    cm_in = (cm >= 0) & (cm <= HC - 1)
    c2 = jnp.where(cm_in, jnp.where(c1_in, cm, c0 - 2 * side), c0 + 2 * side)

    def d2(ci):
        dd = v16 - (16.0 * ci.astype(jnp.float32) + 8.0)
        return dd * dd

    return [(c0, d2(c0)), (c1, d2(c1)), (c2, d2(c2))]


def _floor16(v16):
    t = v16.astype(jnp.int32).astype(jnp.float32)
    return jnp.where(t > v16, t - 1.0, t)


def _sc_body(x_hbm, y_hbm, d2fp_hbm, wsum_hbm, gidx_hbm,
             x_v, y_v, ws_v, idx_v, wt_v, rows_v, gidx_v, sem, sem2):
    wid = lax.axis_index("s") * 2 + lax.axis_index("c")
    base = wid * KPW

    pltpu.sync_copy(x_hbm.at[pl.ds(base, KPW)], x_v)
    pltpu.sync_copy(y_hbm.at[pl.ds(base, KPW)], y_v)

    for h in range(2):
        x16 = x_v[pl.ds(h * 16, 16)]
        y16 = y_v[pl.ds(h * 16, 16)]

        cxs = _axis3(x16)
        cys = _axis3(y16)
        cand = []
        for cyi, dy2 in cys:
            for cxi, dx2 in cxs:
                cand.append((dx2 + dy2, cyi * WC + cxi))
        for k in range(4):
            m = cand[0][0]
            for dcur, _ in cand[1:]:
                m = jnp.minimum(m, dcur)
            f = jnp.full((16,), _BIGI, jnp.int32)
            for dcur, fcur in cand:
                f = jnp.minimum(f, jnp.where(dcur == m, fcur, _BIGI))
            gidx_v[pl.ds(k * KPW + h * 16, 16)] = f
            cand = [(jnp.where(fcur == f, jnp.float32(_BIGF), dcur), fcur)
                    for dcur, fcur in cand]

        ysn = y16 * (1.0 / 16.0) - 0.5
        xsn = x16 * (1.0 / 16.0) - 0.5
        x0 = _floor16(xsn)
        y0 = _floor16(ysn)
        wx1 = xsn - x0
        wx0 = 1.0 - wx1
        wy1 = ysn - y0
        wy0 = 1.0 - wy1
        corners = [(y0, x0, wy0 * wx0), (y0, x0 + 1.0, wy0 * wx1),
                   (y0 + 1.0, x0, wy1 * wx0), (y0 + 1.0, x0 + 1.0, wy1 * wx1)]
        for c, (yi, xi, w) in enumerate(corners):
            valid = ((yi >= 0.0) & (yi <= HC - 1.0)
                     & (xi >= 0.0) & (xi <= WC - 1.0))
            yc = jnp.minimum(jnp.maximum(yi, 0.0), HC - 1.0).astype(jnp.int32)
            xc = jnp.minimum(jnp.maximum(xi, 0.0), WC - 1.0).astype(jnp.int32)
            wt_v[pl.ds(c * KPW + h * 16, 16)] = jnp.where(valid, w, 0.0)
            idx_v[pl.ds(c * KPW + h * 16, 16)] = yc * WC + xc

    gather = pltpu.async_copy(d2fp_hbm.at[idx_v], rows_v, sem)
    gidx_cps = [pltpu.async_copy(gidx_v.at[pl.ds(k * KPW, KPW)],
                                 gidx_hbm.at[pl.ds(k * N + base, KPW)], sem2)
                for k in range(4)]
    gather.wait()

    def kp_body(k, carry):
        wb = []
        for c in range(4):
            wv = wt_v[pl.ds(c * KPW + k, 16)]
            wb.append(jnp.full((16,), 0.0, jnp.float32) + wv[0])
        for j in range(NCH):
            s = wb[0] * rows_v[k, pl.ds(j * 16, 16)]
            for c in range(1, 4):
                s = s + wb[c] * rows_v[c * KPW + k, pl.ds(j * 16, 16)]
            ws_v[pl.ds(k * C + j * 16, 16)] = s
        return carry

    lax.fori_loop(0, KPW, kp_body, 0)

    pltpu.sync_copy(ws_v, wsum_hbm.at[pl.ds(base * C, KPW * C)])
    for cp in gidx_cps:
        cp.wait()


_sc_call = pl.kernel(
    _sc_body,
    mesh=plsc.VectorSubcoreMesh(core_axis_name="c", subcore_axis_name="s"),
    out_type=[
        jax.ShapeDtypeStruct((N * C,), jnp.float32),
        jax.ShapeDtypeStruct((4 * N,), jnp.int32),
    ],
    scratch_types=[
        pltpu.VMEM((KPW,), jnp.float32),
        pltpu.VMEM((KPW,), jnp.float32),
        pltpu.VMEM((KPW * C,), jnp.float32),
        pltpu.VMEM((4 * KPW,), jnp.int32),
        pltpu.VMEM((4 * KPW + 16,), jnp.float32),
        pltpu.VMEM((4 * KPW, CP), jnp.float32),
        pltpu.VMEM((4 * KPW,), jnp.int32),
        pltpu.SemaphoreType.DMA,
        pltpu.SemaphoreType.DMA,
    ],
)


def _tc_body(wsum_ref, gidx_ref, kpd_ref, d2ft_ref, loss_ref, pos_out_ref):
    step = pl.program_id(0)
    A = kpd_ref[...]
    W = wsum_ref[...]
    BT = d2ft_ref[...]

    dot = jnp.sum(A * W, axis=1, keepdims=True)
    nrm = jnp.sqrt(jnp.sum(W * W, axis=1, keepdims=True))
    pos = 2.0 - 2.0 * dot / jnp.maximum(nrm, 1e-12)

    cols = jax.lax.broadcasted_iota(jnp.int32, (R, M), 1)

    S = 2.0 - 2.0 * jax.lax.dot_general(
        A, BT, (((1,), (0,)), ((), ())),
        preferred_element_type=jnp.float32)

    for k in range(4):
        S = jnp.where(cols == gidx_ref[:, k:k + 1], _BIGF, S)

    m1 = S[:, 0:128]
    m2 = jnp.full((R, 128), _BIGF, jnp.float32)
    for g in range(1, 8):
        Sg = S[:, g * 128:(g + 1) * 128]
        m2 = jnp.minimum(m2, jnp.maximum(m1, Sg))
        m1 = jnp.minimum(m1, Sg)

    acc = jnp.float32(0.0)
    for _ in range(4):
        mm = jnp.minimum(jnp.min(m1, axis=1, keepdims=True),
                         jnp.min(m2, axis=1, keepdims=True))
        acc = acc + jnp.sum(jnp.maximum(pos - mm + 1.0, 0.0))
        m1 = jnp.where(m1 == mm, _BIGF, m1)
        m2 = jnp.where(m2 == mm, _BIGF, m2)

    possum = jnp.sum(pos)

    @pl.when(step == 0)
    def _():
        loss_ref[...] = jnp.zeros((1, 1), jnp.float32)
        pos_out_ref[...] = jnp.zeros((1, 1), jnp.float32)

    loss_ref[...] += jnp.reshape(acc, (1, 1))
    pos_out_ref[...] += jnp.reshape(possum, (1, 1))

    @pl.when(step == GRID_STEPS - 1)
    def _():
        loss_ref[...] = loss_ref[...] * (1.0 / (4.0 * N))
        pos_out_ref[...] = pos_out_ref[...] * (1.0 / N)


@jax.jit
def _run(x, y, kp1_desc, d2fp, d2ft):
    wsum, gidx = _sc_call(x, y, d2fp)
    loss, posmean = pl.pallas_call(
        _tc_body,
        grid=(GRID_STEPS,),
        in_specs=[
            pl.BlockSpec((R, C), lambda i: (i, 0)),
            pl.BlockSpec((R, 4), lambda i: (i, 0)),
            pl.BlockSpec((R, C), lambda i: (i, 0)),
            pl.BlockSpec((C, M), lambda i: (0, 0)),
        ],
        out_specs=[
            pl.BlockSpec((1, 1), lambda i: (0, 0)),
            pl.BlockSpec((1, 1), lambda i: (0, 0)),
        ],
        out_shape=[
            jax.ShapeDtypeStruct((1, 1), jnp.float32),
            jax.ShapeDtypeStruct((1, 1), jnp.float32),
        ],
    )(wsum.reshape(N, C), gidx.reshape(4, N).T, kp1_desc, d2ft)
    return loss[0, 0], posmean[0, 0]


def kernel(kp1, w_kp1, kp1_desc, desc2):
    d2ft = desc2[0].reshape(C, M)
    d2f = jnp.transpose(d2ft, (1, 0))
    d2fp = jnp.pad(d2f, ((0, 0), (0, CP - C)))
    x = w_kp1[:, 1]
    y = w_kp1[:, 0]
    return _run(x, y, kp1_desc, d2fp, d2ft)


# speedup vs baseline: 1.0637x; 1.0046x over previous
import jax
import jax.numpy as jnp
from jax import lax
from jax.experimental import pallas as pl
from jax.experimental.pallas import tpu as pltpu
from jax.experimental.pallas import tpu_sc as plsc

N = 1024
C = 192
CP = 256
HC = 32
WC = 32
M = HC * WC
R = 512
GRID_STEPS = N // R

NW = 32
KPW = N // NW
NCH = C // 16

_BIGF = 1e30
_BIGI = 2**30


def _axis3(v16):
    c0 = (v16 * (1.0 / 16.0)).astype(jnp.int32)
    c0f = c0.astype(jnp.float32)
    d0 = v16 - (16.0 * c0f + 8.0)
    side = jnp.where(d0 >= 0.0, 1, -1).astype(jnp.int32)
    c1 = c0 + side
    c1_in = (c1 >= 0) & (c1 <= HC - 1)
    c1 = jnp.where(c1_in, c1, c0 - side)
    cm = c0 - side
    cm_in = (cm >= 0) & (cm <= HC - 1)
    c2 = jnp.where(cm_in, jnp.where(c1_in, cm, c0 - 2 * side), c0 + 2 * side)

    def d2(ci):
        dd = v16 - (16.0 * ci.astype(jnp.float32) + 8.0)
        return dd * dd

    return [(c0, d2(c0)), (c1, d2(c1)), (c2, d2(c2))]


def _floor16(v16):
    t = v16.astype(jnp.int32).astype(jnp.float32)
    return jnp.where(t > v16, t - 1.0, t)


def _sc_body(x_hbm, y_hbm, d2fp_hbm, wsum_hbm, gidx_hbm,
             x_v, y_v, ws_v, idx_v, wt_v, rows_v, gidx_v, sem, sem2):
    wid = lax.axis_index("s") * 2 + lax.axis_index("c")
    base = wid * KPW

    pltpu.sync_copy(x_hbm.at[pl.ds(base, KPW)], x_v)
    pltpu.sync_copy(y_hbm.at[pl.ds(base, KPW)], y_v)

    for h in range(2):
        x16 = x_v[pl.ds(h * 16, 16)]
        y16 = y_v[pl.ds(h * 16, 16)]
        ysn = y16 * (1.0 / 16.0) - 0.5
        xsn = x16 * (1.0 / 16.0) - 0.5
        x0 = _floor16(xsn)
        y0 = _floor16(ysn)
        wx1 = xsn - x0
        wx0 = 1.0 - wx1
        wy1 = ysn - y0
        wy0 = 1.0 - wy1
        corners = [(y0, x0, wy0 * wx0), (y0, x0 + 1.0, wy0 * wx1),
                   (y0 + 1.0, x0, wy1 * wx0), (y0 + 1.0, x0 + 1.0, wy1 * wx1)]
        for c, (yi, xi, w) in enumerate(corners):
            valid = ((yi >= 0.0) & (yi <= HC - 1.0)
                     & (xi >= 0.0) & (xi <= WC - 1.0))
            yc = jnp.minimum(jnp.maximum(yi, 0.0), HC - 1.0).astype(jnp.int32)
            xc = jnp.minimum(jnp.maximum(xi, 0.0), WC - 1.0).astype(jnp.int32)
            wt_v[pl.ds(c * KPW + h * 16, 16)] = jnp.where(valid, w, 0.0)
            idx_v[pl.ds(c * KPW + h * 16, 16)] = yc * WC + xc

    gather = pltpu.async_copy(d2fp_hbm.at[idx_v], rows_v, sem)

    for h in range(2):
        x16 = x_v[pl.ds(h * 16, 16)]
        y16 = y_v[pl.ds(h * 16, 16)]

        cxs = _axis3(x16)
        cys = _axis3(y16)
        cand = []
        for cyi, dy2 in cys:
            for cxi, dx2 in cxs:
                cand.append((dx2 + dy2, cyi * WC + cxi))
        for k in range(4):
            m = cand[0][0]
            for dcur, _ in cand[1:]:
                m = jnp.minimum(m, dcur)
            f = jnp.full((16,), _BIGI, jnp.int32)
            for dcur, fcur in cand:
                f = jnp.minimum(f, jnp.where(dcur == m, fcur, _BIGI))
            gidx_v[pl.ds(k * KPW + h * 16, 16)] = f
            cand = [(jnp.where(fcur == f, jnp.float32(_BIGF), dcur), fcur)
                    for dcur, fcur in cand]

    gidx_cps = [pltpu.async_copy(gidx_v.at[pl.ds(k * KPW, KPW)],
                                 gidx_hbm.at[pl.ds(k * N + base, KPW)], sem2)
                for k in range(4)]
    gather.wait()

    def kp_body(k, carry):
        wb = []
        for c in range(4):
            wv = wt_v[pl.ds(c * KPW + k, 16)]
            wb.append(jnp.full((16,), 0.0, jnp.float32) + wv[0])
        for j in range(NCH):
            s = wb[0] * rows_v[k, pl.ds(j * 16, 16)]
            for c in range(1, 4):
                s = s + wb[c] * rows_v[c * KPW + k, pl.ds(j * 16, 16)]
            ws_v[pl.ds(k * C + j * 16, 16)] = s
        return carry

    lax.fori_loop(0, KPW, kp_body, 0)

    pltpu.sync_copy(ws_v, wsum_hbm.at[pl.ds(base * C, KPW * C)])
    for cp in gidx_cps:
        cp.wait()


_sc_call = pl.kernel(
    _sc_body,
    mesh=plsc.VectorSubcoreMesh(core_axis_name="c", subcore_axis_name="s"),
    out_type=[
        jax.ShapeDtypeStruct((N * C,), jnp.float32),
        jax.ShapeDtypeStruct((4 * N,), jnp.int32),
    ],
    scratch_types=[
        pltpu.VMEM((KPW,), jnp.float32),
        pltpu.VMEM((KPW,), jnp.float32),
        pltpu.VMEM((KPW * C,), jnp.float32),
        pltpu.VMEM((4 * KPW,), jnp.int32),
        pltpu.VMEM((4 * KPW + 16,), jnp.float32),
        pltpu.VMEM((4 * KPW, CP), jnp.float32),
        pltpu.VMEM((4 * KPW,), jnp.int32),
        pltpu.SemaphoreType.DMA,
        pltpu.SemaphoreType.DMA,
    ],
)


def _tc_body(wsum_ref, gidx_ref, kpd_ref, d2ft_ref, loss_ref, pos_out_ref):
    step = pl.program_id(0)
    A = kpd_ref[...]
    W = wsum_ref[...]
    BT = d2ft_ref[...]

    dot = jnp.sum(A * W, axis=1, keepdims=True)
    nrm = jnp.sqrt(jnp.sum(W * W, axis=1, keepdims=True))
    pos = 2.0 - 2.0 * dot / jnp.maximum(nrm, 1e-12)

    cols = jax.lax.broadcasted_iota(jnp.int32, (R, M), 1)

    S = 2.0 - 2.0 * jax.lax.dot_general(
        A, BT, (((1,), (0,)), ((), ())),
        preferred_element_type=jnp.float32)

    for k in range(4):
        S = jnp.where(cols == gidx_ref[:, k:k + 1], _BIGF, S)

    m1 = S[:, 0:128]
    m2 = jnp.full((R, 128), _BIGF, jnp.float32)
    for g in range(1, 8):
        Sg = S[:, g * 128:(g + 1) * 128]
        m2 = jnp.minimum(m2, jnp.maximum(m1, Sg))
        m1 = jnp.minimum(m1, Sg)

    acc = jnp.float32(0.0)
    for _ in range(4):
        mm = jnp.minimum(jnp.min(m1, axis=1, keepdims=True),
                         jnp.min(m2, axis=1, keepdims=True))
        acc = acc + jnp.sum(jnp.maximum(pos - mm + 1.0, 0.0))
        m1 = jnp.where(m1 == mm, _BIGF, m1)
        m2 = jnp.where(m2 == mm, _BIGF, m2)

    possum = jnp.sum(pos)

    @pl.when(step == 0)
    def _():
        loss_ref[...] = jnp.zeros((1, 1), jnp.float32)
        pos_out_ref[...] = jnp.zeros((1, 1), jnp.float32)

    loss_ref[...] += jnp.reshape(acc, (1, 1))
    pos_out_ref[...] += jnp.reshape(possum, (1, 1))

    @pl.when(step == GRID_STEPS - 1)
    def _():
        loss_ref[...] = loss_ref[...] * (1.0 / (4.0 * N))
        pos_out_ref[...] = pos_out_ref[...] * (1.0 / N)


@jax.jit
def _run(x, y, kp1_desc, d2fp, d2ft):
    wsum, gidx = _sc_call(x, y, d2fp)
    loss, posmean = pl.pallas_call(
        _tc_body,
        grid=(GRID_STEPS,),
        in_specs=[
            pl.BlockSpec((R, C), lambda i: (i, 0)),
            pl.BlockSpec((R, 4), lambda i: (i, 0)),
            pl.BlockSpec((R, C), lambda i: (i, 0)),
            pl.BlockSpec((C, M), lambda i: (0, 0)),
        ],
        out_specs=[
            pl.BlockSpec((1, 1), lambda i: (0, 0)),
            pl.BlockSpec((1, 1), lambda i: (0, 0)),
        ],
        out_shape=[
            jax.ShapeDtypeStruct((1, 1), jnp.float32),
            jax.ShapeDtypeStruct((1, 1), jnp.float32),
        ],
    )(wsum.reshape(N, C), gidx.reshape(4, N).T, kp1_desc, d2ft)
    return loss[0, 0], posmean[0, 0]


def kernel(kp1, w_kp1, kp1_desc, desc2):
    d2ft = desc2[0].reshape(C, M)
    d2f = jnp.transpose(d2ft, (1, 0))
    d2fp = jnp.pad(d2f, ((0, 0), (0, CP - C)))
    x = w_kp1[:, 1]
    y = w_kp1[:, 0]
    return _run(x, y, kp1_desc, d2fp, d2ft)
